# Initial kernel scaffold; baseline (speedup 1.0000x reference)
import jax
import jax.numpy as jnp
from jax.experimental import pallas as pl


def kernel(pcp_batch, pcp_masks, table, W, b):
    raise NotImplementedError("write your pallas kernel here")


# speedup vs baseline: 3.4699x; 3.4699x over previous
import functools

import jax
import jax.numpy as jnp
from jax import lax
from jax.experimental import pallas as pl
from jax.experimental.pallas import tpu as pltpu
from jax.experimental.pallas import tpu_sc as plsc

_PAD = 39972

_NC = 2
_NS = 16
_NW = _NC * _NS

_CH = 128

_ROW_BLK = 1024


def _proj_body(tab_ref, w_ref, b_ref, out_ref):
    i = pl.program_id(0)
    row = i * _ROW_BLK + lax.broadcasted_iota(jnp.int32, (_ROW_BLK, 1), 0)
    t = jnp.where(row != _PAD, tab_ref[...], 0.0)
    y = jnp.dot(t, w_ref[...], preferred_element_type=jnp.float32)
    out_ref[...] = jnp.maximum(y + b_ref[...], 0.0)


def _project_table(table, W, b):
    V, D = table.shape
    H = W.shape[1]
    grid = pl.cdiv(V, _ROW_BLK)
    return pl.pallas_call(
        _proj_body,
        grid=(grid,),
        in_specs=[
            pl.BlockSpec((_ROW_BLK, D), lambda i: (i, 0)),
            pl.BlockSpec((D, H), lambda i: (0, 0)),
            pl.BlockSpec((1, H), lambda i: (0, 0)),
        ],
        out_specs=pl.BlockSpec((_ROW_BLK, H), lambda i: (i, 0)),
        out_shape=jax.ShapeDtypeStruct((V, H), jnp.float32),
    )(table, W, b.reshape(1, H))


def _make_gather(B, H):
    assert B % (_NW * _CH) == 0
    bpw = B // _NW
    nchunk = bpw // _CH

    mesh = plsc.VectorSubcoreMesh(
        core_axis_name="c", subcore_axis_name="s",
        num_cores=_NC, num_subcores=_NS,
    )

    @functools.partial(
        pl.kernel,
        out_type=jax.ShapeDtypeStruct((B, H), jnp.float32),
        mesh=mesh,
        compiler_params=pltpu.CompilerParams(use_tc_tiling_on_sc=False),
        scratch_types=[
            pltpu.VMEM((bpw,), jnp.int32),
            pltpu.VMEM((_CH, H), jnp.float32),
            pltpu.SemaphoreType.DMA,
        ],
    )
    def gather(ytab_hbm, idx_hbm, out_hbm, idx_v, rows_v, gsem):
        wid = lax.axis_index("s") * _NC + lax.axis_index("c")
        base = wid * bpw
        pltpu.sync_copy(idx_hbm.at[pl.ds(base, bpw)], idx_v)

        def body(j, carry):
            off = j * _CH
            pltpu.async_copy(
                ytab_hbm.at[idx_v.at[pl.ds(off, _CH)]], rows_v, gsem,
            ).wait()
            pltpu.sync_copy(rows_v, out_hbm.at[pl.ds(base + off, _CH)])
            return carry

        lax.fori_loop(0, nchunk, body, 0)

    return gather


def kernel(pcp_batch, pcp_masks, table, W, b):
    n, s = pcp_batch.shape
    H = W.shape[1]
    ytab = _project_table(table, W, b)
    idx = pcp_batch.reshape(-1).astype(jnp.int32)
    y = _make_gather(n * s, H)(ytab, idx)
    return y.reshape(n, s, H), pcp_masks


# speedup vs baseline: 3.8586x; 1.1120x over previous
import functools

import jax
import jax.numpy as jnp
from jax import lax
from jax.experimental import pallas as pl
from jax.experimental.pallas import tpu as pltpu
from jax.experimental.pallas import tpu_sc as plsc

_PAD = 39972

_NC = 2
_NS = 16
_NW = _NC * _NS

_CH = 128

_ROW_BLK = 1024


def _proj_body(tab_ref, w_ref, b_ref, out_ref):
    i = pl.program_id(0)
    row = i * _ROW_BLK + lax.broadcasted_iota(jnp.int32, (_ROW_BLK, 1), 0)
    t = jnp.where(row != _PAD, tab_ref[...], 0.0)
    y = jnp.dot(t, w_ref[...], preferred_element_type=jnp.float32)
    out_ref[...] = jnp.maximum(y + b_ref[...], 0.0)


def _project_table(table, W, b):
    V, D = table.shape
    H = W.shape[1]
    Wp = jnp.pad(W, ((0, 0), (0, D - H)))
    bp = jnp.pad(b, (0, D - H)).reshape(1, D)
    grid = pl.cdiv(V, _ROW_BLK)
    return pl.pallas_call(
        _proj_body,
        grid=(grid,),
        in_specs=[
            pl.BlockSpec((_ROW_BLK, D), lambda i: (i, 0)),
            pl.BlockSpec((D, D), lambda i: (0, 0)),
            pl.BlockSpec((1, D), lambda i: (0, 0)),
        ],
        out_specs=pl.BlockSpec((_ROW_BLK, D), lambda i: (i, 0)),
        out_shape=jax.ShapeDtypeStruct((V, D), jnp.float32),
    )(table, Wp, bp)


def _make_gather(B, D, H):
    assert B % (_NW * _CH) == 0
    bpw = B // _NW
    nchunk = bpw // _CH

    mesh = plsc.VectorSubcoreMesh(
        core_axis_name="c", subcore_axis_name="s",
        num_cores=_NC, num_subcores=_NS,
    )

    @functools.partial(
        pl.kernel,
        out_type=jax.ShapeDtypeStruct((B, H), jnp.float32),
        mesh=mesh,
        scratch_types=[
            pltpu.VMEM((bpw,), jnp.int32),
            pltpu.VMEM((_CH, D), jnp.float32),
            pltpu.VMEM((_CH, H), jnp.float32),
            pltpu.SemaphoreType.DMA,
        ],
    )
    def gather(ytab_hbm, idx_hbm, out_hbm, idx_v, rows_v, pack_v, gsem):
        wid = lax.axis_index("s") * _NC + lax.axis_index("c")
        base = wid * bpw
        pltpu.sync_copy(idx_hbm.at[pl.ds(base, bpw)], idx_v)

        def body(j, carry):
            off = j * _CH
            pltpu.async_copy(
                ytab_hbm.at[idx_v.at[pl.ds(off, _CH)]], rows_v, gsem,
            ).wait()

            def compact(r, c2):
                for c in range(H // 16):
                    pack_v[r, pl.ds(c * 16, 16)] = rows_v[r, pl.ds(c * 16, 16)]
                return c2

            lax.fori_loop(0, _CH, compact, 0)
            pltpu.sync_copy(pack_v, out_hbm.at[pl.ds(base + off, _CH)])
            return carry

        lax.fori_loop(0, nchunk, body, 0)

    return gather


def kernel(pcp_batch, pcp_masks, table, W, b):
    n, s = pcp_batch.shape
    H = W.shape[1]
    ytab = _project_table(table, W, b)
    idx = pcp_batch.reshape(-1).astype(jnp.int32)
    y = _make_gather(n * s, table.shape[1], H)(ytab, idx)
    return y.reshape(n, s, H), pcp_masks
